# baseline (speedup 1.0000x reference)
import functools

import jax
import jax.numpy as jnp
from jax import lax
from jax.experimental import pallas as pl
from jax.experimental.pallas import tpu as pltpu
from jax.experimental.pallas import tpu_sc as plsc

BATCH = 16384
CODES = 8192
DIM = 32

BM = 256
HALF = CODES // 2

NC = 2
NS = 16
NW = NC * NS
B_PER_W = BATCH // NW


RC = 64
NT = HALF // 128


def _argmin_body(x_ref, w_ref, out_ref, cl_ref, c2_ref):
    i = pl.program_id(0)

    @pl.when(i == 0)
    def _():
        wall = w_ref[...]
        c2_ref[...] = jnp.sum(wall * wall, axis=1)[None, :]

    xb = x_ref[...]
    l2 = jnp.sum(xb * xb, axis=1, keepdims=True)
    lane = lax.broadcasted_iota(jnp.int32, (RC, 128), 1).astype(jnp.float32)
    per_half = []
    for h in range(2):
        wb = w_ref[h * HALF:(h + 1) * HALF, :]
        cl_ref[...] = lax.dot_general(xb, wb, (((1,), (1,)), ((), ())),
                                      preferred_element_type=jnp.float32)
        chunks = []
        for rc in range(BM // RC):
            l2c = l2[rc * RC:(rc + 1) * RC, :]

            def step(t, carry, _rc=rc, _h=h, _l2c=l2c):
                minv, mini = carry
                clt = cl_ref[pl.ds(_rc * RC, RC), pl.ds(t * 128, 128)]
                c2t = c2_ref[:, pl.ds(_h * HALF + t * 128, 128)]
                dt = _l2c - 2.0 * clt + c2t
                upd = dt < minv
                return (jnp.where(upd, dt, minv),
                        jnp.where(upd, t.astype(jnp.float32), mini))

            minv, mini = lax.fori_loop(
                0, NT, step,
                (jnp.full((RC, 128), jnp.inf, jnp.float32),
                 jnp.zeros((RC, 128), jnp.float32)))
            rowmin = jnp.min(minv, axis=1, keepdims=True)
            jj = jnp.where(minv == rowmin, mini * 128.0 + lane,
                           jnp.float32(1e9))
            rowarg = jnp.min(jj, axis=1, keepdims=True)
            chunks.append((rowmin, rowarg))
        per_half.append(chunks)

    for rc in range(BM // RC):
        v0, i0 = per_half[0][rc]
        v1, i1 = per_half[1][rc]
        v0r = v0.astype(jnp.bfloat16).astype(jnp.float32)
        res = jnp.where(v1 < v0r, i1 + float(HALF), i0)
        out_ref[pl.ds(rc * RC, RC), :] = res.astype(jnp.int32)


def _closest_indices(x, W):
    return pl.pallas_call(
        _argmin_body,
        grid=(BATCH // BM,),
        in_specs=[
            pl.BlockSpec((BM, DIM), lambda i: (i, 0)),
            pl.BlockSpec((CODES, DIM), lambda i: (0, 0)),
        ],
        out_specs=pl.BlockSpec((BM, 1), lambda i: (i, 0)),
        out_shape=jax.ShapeDtypeStruct((BATCH, 1), jnp.int32),
        scratch_shapes=[
            pltpu.VMEM((BM, HALF), jnp.float32),
            pltpu.VMEM((1, CODES), jnp.float32),
        ],
    )(x, W)


@functools.cache
def _make_sc_gather():
    @functools.partial(
        pl.kernel,
        mesh=plsc.VectorSubcoreMesh(core_axis_name="c", subcore_axis_name="s"),
        out_type=jax.ShapeDtypeStruct((BATCH, DIM), jnp.float32),
        scratch_types=[
            pltpu.VMEM((B_PER_W,), jnp.int32),
            pltpu.VMEM((B_PER_W, DIM), jnp.float32),
            pltpu.SemaphoreType.DMA,
        ],
        compiler_params=pltpu.CompilerParams(use_tc_tiling_on_sc=False),
    )
    def _sc_gather(table_hbm, idx_hbm, out_hbm, idx_v, rows_v, sem):
        wid = lax.axis_index("s") * NC + lax.axis_index("c")
        base = wid * B_PER_W
        pltpu.sync_copy(idx_hbm.at[pl.ds(base, B_PER_W)], idx_v)
        pltpu.async_copy(table_hbm.at[idx_v], rows_v, sem).wait()
        pltpu.sync_copy(rows_v, out_hbm.at[pl.ds(base, B_PER_W)])

    return _sc_gather


def kernel(x, W):
    closest = _closest_indices(x, W).reshape(BATCH)
    return _make_sc_gather()(W, closest)


# speedup vs baseline: 2.0224x; 2.0224x over previous
import functools

import jax
import jax.numpy as jnp
from jax import lax
from jax.experimental import pallas as pl
from jax.experimental.pallas import tpu as pltpu
from jax.experimental.pallas import tpu_sc as plsc

BATCH = 16384
CODES = 8192
DIM = 32

BM = 1024
HALF = CODES // 2

NC = 2
NS = 16
NW = NC * NS
B_PER_W = BATCH // NW


def _half_argmin(xm2, l2, wb):
    cl2 = lax.dot_general(xm2, wb, (((1,), (1,)), ((), ())),
                          preferred_element_type=jnp.float32)
    c2 = jnp.sum(wb * wb, axis=1)[None, :]
    d = (l2 + cl2) + c2
    lmin = jnp.min(d, axis=1, keepdims=True)
    ids = lax.broadcasted_iota(jnp.int32, d.shape, 1)
    larg = jnp.min(jnp.where(d == lmin, ids, jnp.int32(2 ** 30)),
                   axis=1, keepdims=True)
    return lmin, larg


def _argmin_body(x_ref, w_ref, out_ref):
    xb = x_ref[...]
    l2 = jnp.sum(xb * xb, axis=1, keepdims=True)
    xm2 = xb * (-2.0)
    v0, i0 = _half_argmin(xm2, l2, w_ref[0:HALF, :])
    v1, i1 = _half_argmin(xm2, l2, w_ref[HALF:CODES, :])
    v0r = v0.astype(jnp.bfloat16).astype(jnp.float32)
    out_ref[...] = jnp.where(v1 < v0r, i1 + HALF, i0)


def _closest_indices(x, W):
    return pl.pallas_call(
        _argmin_body,
        grid=(BATCH // BM,),
        in_specs=[
            pl.BlockSpec((BM, DIM), lambda i: (i, 0)),
            pl.BlockSpec((CODES, DIM), lambda i: (0, 0)),
        ],
        out_specs=pl.BlockSpec((BM, 1), lambda i: (i, 0)),
        out_shape=jax.ShapeDtypeStruct((BATCH, 1), jnp.int32),
    )(x, W)


@functools.cache
def _make_sc_gather():
    @functools.partial(
        pl.kernel,
        mesh=plsc.VectorSubcoreMesh(core_axis_name="c", subcore_axis_name="s"),
        out_type=jax.ShapeDtypeStruct((BATCH, DIM), jnp.float32),
        scratch_types=[
            pltpu.VMEM((B_PER_W,), jnp.int32),
            pltpu.VMEM((B_PER_W, DIM), jnp.float32),
            pltpu.SemaphoreType.DMA,
        ],
        compiler_params=pltpu.CompilerParams(use_tc_tiling_on_sc=False),
    )
    def _sc_gather(table_hbm, idx_hbm, out_hbm, idx_v, rows_v, sem):
        wid = lax.axis_index("s") * NC + lax.axis_index("c")
        base = wid * B_PER_W
        pltpu.sync_copy(idx_hbm.at[pl.ds(base, B_PER_W)], idx_v)
        pltpu.async_copy(table_hbm.at[idx_v], rows_v, sem).wait()
        pltpu.sync_copy(rows_v, out_hbm.at[pl.ds(base, B_PER_W)])

    return _sc_gather


def kernel(x, W):
    closest = _closest_indices(x, W).reshape(BATCH)
    return _make_sc_gather()(W, closest)


# speedup vs baseline: 2.2384x; 1.1068x over previous
import functools

import jax
import jax.numpy as jnp
from jax import lax
from jax.experimental import pallas as pl
from jax.experimental.pallas import tpu as pltpu
from jax.experimental.pallas import tpu_sc as plsc

BATCH = 16384
CODES = 8192
DIM = 32

BM = 1024
HALF = CODES // 2

NC = 2
NS = 16
NW = NC * NS
B_PER_W = BATCH // NW


def _half_argmin(xm2, l2, wb):
    cl2 = lax.dot_general(xm2, wb, (((1,), (1,)), ((), ())),
                          preferred_element_type=jnp.float32)
    c2 = jnp.sum(wb * wb, axis=1)[None, :]
    d = (l2 + cl2) + c2
    lmin = jnp.min(d, axis=1, keepdims=True)
    ids = lax.broadcasted_iota(jnp.int32, d.shape, 1).astype(jnp.float32)
    larg = jnp.min(jnp.where(d == lmin, ids, jnp.float32(1e9)),
                   axis=1, keepdims=True)
    return lmin, larg


def _argmin_body(x_ref, w_ref, out_ref):
    xb = x_ref[...]
    l2 = jnp.sum(xb * xb, axis=1, keepdims=True)
    xm2 = xb * (-2.0)
    v0, i0 = _half_argmin(xm2, l2, w_ref[0:HALF, :])
    v1, i1 = _half_argmin(xm2, l2, w_ref[HALF:CODES, :])
    v0r = v0.astype(jnp.bfloat16).astype(jnp.float32)
    out_ref[...] = jnp.where(v1 < v0r, i1 + float(HALF), i0).astype(jnp.int32)


def _closest_indices(x, W):
    return pl.pallas_call(
        _argmin_body,
        grid=(BATCH // BM,),
        in_specs=[
            pl.BlockSpec((BM, DIM), lambda i: (i, 0)),
            pl.BlockSpec((CODES, DIM), lambda i: (0, 0)),
        ],
        out_specs=pl.BlockSpec((BM, 1), lambda i: (i, 0)),
        out_shape=jax.ShapeDtypeStruct((BATCH, 1), jnp.int32),
    )(x, W)


@functools.cache
def _make_sc_gather():
    @functools.partial(
        pl.kernel,
        mesh=plsc.VectorSubcoreMesh(core_axis_name="c", subcore_axis_name="s"),
        out_type=jax.ShapeDtypeStruct((BATCH, DIM), jnp.float32),
        scratch_types=[
            pltpu.VMEM((B_PER_W,), jnp.int32),
            pltpu.VMEM((B_PER_W, DIM), jnp.float32),
            pltpu.SemaphoreType.DMA,
        ],
        compiler_params=pltpu.CompilerParams(use_tc_tiling_on_sc=False),
    )
    def _sc_gather(table_hbm, idx_hbm, out_hbm, idx_v, rows_v, sem):
        wid = lax.axis_index("s") * NC + lax.axis_index("c")
        base = wid * B_PER_W
        pltpu.sync_copy(idx_hbm.at[pl.ds(base, B_PER_W)], idx_v)
        pltpu.async_copy(table_hbm.at[idx_v], rows_v, sem).wait()
        pltpu.sync_copy(rows_v, out_hbm.at[pl.ds(base, B_PER_W)])

    return _sc_gather


def kernel(x, W):
    closest = _closest_indices(x, W).reshape(BATCH)
    return _make_sc_gather()(W, closest)
